# Initial kernel scaffold; baseline (speedup 1.0000x reference)
import jax
import jax.numpy as jnp
from jax.experimental import pallas as pl


def kernel(fea_r_x4, fea_r_x2, fea_r_x1, fea_ref_x4, fea_ref_x2, fea_ref_x1):
    raise NotImplementedError("write your pallas kernel here")


# speedup vs baseline: 25.0450x; 25.0450x over previous
import functools

import numpy as np
import jax
import jax.numpy as jnp
from jax.experimental import pallas as pl
from jax.experimental.pallas import tpu as pltpu

F32 = jnp.float32


def _coarse_kernel(q_ref, refpad_ref, out_ref, *, H, W):
    B = q_ref.shape[2]
    C = q_ref.shape[3]
    HW = H * W
    corr = jnp.zeros((HW, B), dtype=F32)
    for di, d in enumerate((1, 2, 3)):
        qd = q_ref[di]
        qsq = jnp.sum(jnp.sum(qd * qd, axis=2), axis=0, keepdims=True)
        qinv = 1.0 / (jnp.sqrt(qsq) + 1e-6)
        acc = jnp.zeros((HW, B), dtype=F32)
        nsq = jnp.zeros((HW, 1), dtype=F32)
        for i in range(3):
            for j in range(3):
                r0 = 3 + (i - 1) * d
                c0 = 3 + (j - 1) * d
                sl = refpad_ref[r0:r0 + H, c0:c0 + W, :].reshape(HW, C)
                acc = acc + jax.lax.dot_general(
                    sl, qd[i * 3 + j],
                    dimension_numbers=(((1,), (1,)), ((), ())),
                    preferred_element_type=F32)
                nsq = nsq + jnp.sum(sl * sl, axis=1, keepdims=True)
        pinv = 1.0 / (jnp.sqrt(nsq) + 1e-6)
        corr = corr + acc * pinv * qinv
    idx = jnp.argmax(corr, axis=0).astype(jnp.int32)
    idxh = idx // W
    idxw = idx % W
    h1 = jnp.clip(idxh - 7, 0, H - 15)
    w1 = jnp.clip(idxw - 7, 0, W - 15)
    pad = jnp.zeros((128 - B,), jnp.int32)
    rows = jnp.concatenate([
        jnp.concatenate([h1, pad])[None, :],
        jnp.concatenate([w1, pad])[None, :],
        jnp.zeros((6, 128), jnp.int32),
    ], axis=0)
    out_ref[...] = rows


def _fine_kernel(pos_ref, rpad_ref, refc_ref, idx_out_ref, att_out_ref, *,
                 nbw, W):
    b = pl.program_id(0)
    bh = b // nbw
    bw = b % nbw
    C = refc_ref.shape[2]
    nc = W - 2
    ih = pos_ref[0, b]
    iw = pos_ref[1, b]
    band = refc_ref[pl.ds(ih, 15), :, :]
    rblk = rpad_ref[pl.ds(bh * 8, 10), pl.ds(bw * 8, 10), :]
    refp = jnp.concatenate(
        [band[i:i + 13, j:j + nc, :].reshape(13 * nc, C)
         for i in range(3) for j in range(3)], axis=1)
    rp = jnp.concatenate(
        [rblk[i:i + 8, j:j + 8, :].reshape(64, C)
         for i in range(3) for j in range(3)], axis=1)
    rinv = 1.0 / (jnp.sqrt(jnp.sum(rp * rp, axis=1, keepdims=True)) + 1e-6)
    finv = 1.0 / (jnp.sqrt(jnp.sum(refp * refp, axis=1, keepdims=True)) + 1e-6)
    corr = jax.lax.dot_general(
        rp * rinv, refp * finv,
        dimension_numbers=(((1,), (1,)), ((), ())),
        preferred_element_type=F32)
    pw = jax.lax.broadcasted_iota(jnp.int32, (64, 13 * nc), 1) % nc
    valid = (pw >= iw) & (pw < iw + 13)
    corr = jnp.where(valid, corr, -3.0e38)
    am = jnp.argmax(corr, axis=1).astype(jnp.int32)
    idx_out_ref[0, 0, :] = (am // nc) * 13 + (am % nc) - iw
    att_out_ref[0, 0, :] = jnp.max(corr, axis=1)


def _transfer_kernel(pos_ref, idx_ref, shifts_ref, invcnt_ref, out_ref,
                     acc_ref, *, s):
    b = pl.program_id(0)
    C = shifts_ref.shape[3]
    ih = pos_ref[0, b]
    iw = pos_ref[1, b]
    acc_ref[...] = jnp.zeros((10 * s, 10 * s, C), F32)
    for p in range(64):
        q = idx_ref[b, p]
        qh = q // 13
        qw = q % 13
        rs = (ih + qh) * s
        cs = (iw + qw) * s
        a = (cs // 8) * 8
        k = (cs - a) // s
        sel = shifts_ref[pl.ds(k, 1), pl.ds(rs, 3 * s), pl.ds(a, 3 * s), :][0]
        gh, gw = p // 8, p % 8
        acc_ref[gh * s:gh * s + 3 * s, gw * s:gw * s + 3 * s, :] += sel
    out_ref[0] = acc_ref[s:9 * s, s:9 * s, :] * invcnt_ref[0]


def _fold_inv_count(s):
    ps = 3 * s
    cnt = np.zeros((10 * s, 10 * s), np.float32)
    for gh in range(8):
        for gw in range(8):
            cnt[gh * s:gh * s + ps, gw * s:gw * s + ps] += 1.0
    cnt = np.maximum(cnt[s:9 * s, s:9 * s], 1.0)
    return (1.0 / cnt).astype(np.float32)


def kernel(fea_r_x4, fea_r_x2, fea_r_x1, fea_ref_x4, fea_ref_x2, fea_ref_x1):
    N, C1, Hr, Wr = fea_r_x1.shape
    C2 = fea_r_x2.shape[1]
    C4 = fea_r_x4.shape[1]
    nb = Hr // 8
    B = nb * nb

    fr1c = jnp.transpose(fea_r_x1[0], (1, 2, 0))
    ff1c = jnp.transpose(fea_ref_x1[0], (1, 2, 0))
    ff2c = jnp.transpose(fea_ref_x2[0], (1, 2, 0))
    ff4c = jnp.transpose(fea_ref_x4[0], (1, 2, 0))

    refpad = jnp.pad(ff1c, ((3, 3), (3, 3), (0, 0)), mode='edge')
    rpad = jnp.pad(fr1c, ((1, 1), (1, 1), (0, 0)), mode='edge')

    ctr = jnp.arange(nb) * 8 + 4
    qs = []
    for d in (1, 2, 3):
        per_d = []
        for i in range(3):
            for j in range(3):
                sl = fr1c[ctr[:, None] + (i - 1) * d,
                          ctr[None, :] + (j - 1) * d, :]
                per_d.append(sl.reshape(B, C1))
        qs.append(jnp.stack(per_d))
    q = jnp.stack(qs)

    posout = pl.pallas_call(
        functools.partial(_coarse_kernel, H=Hr, W=Wr),
        out_shape=jax.ShapeDtypeStruct((8, 128), jnp.int32),
    )(q, refpad)
    pos = posout[:2, :B]

    fine_grid = pltpu.PrefetchScalarGridSpec(
        num_scalar_prefetch=1,
        grid=(B,),
        in_specs=[
            pl.BlockSpec(rpad.shape, lambda b, pos_r: (0, 0, 0)),
            pl.BlockSpec(ff1c.shape, lambda b, pos_r: (0, 0, 0)),
        ],
        out_specs=[
            pl.BlockSpec((1, 1, 64), lambda b, pos_r: (b, 0, 0)),
            pl.BlockSpec((1, 1, 64), lambda b, pos_r: (b, 0, 0)),
        ],
    )
    idx_all, att = pl.pallas_call(
        functools.partial(_fine_kernel, nbw=nb, W=Wr),
        grid_spec=fine_grid,
        out_shape=[jax.ShapeDtypeStruct((B, 1, 64), jnp.int32),
                   jax.ShapeDtypeStruct((B, 1, 64), F32)],
    )(pos, rpad, ff1c)
    idx_flat = idx_all[:, 0, :]

    warped = {}
    for s, refc, C in ((1, ff1c, C1), (2, ff2c, C2), (4, ff4c, C4)):
        ncopy = 8 // s
        Hs, Ws = refc.shape[0], refc.shape[1]
        shifts = jnp.stack(
            [jnp.pad(refc[:, k * s:, :], ((0, 0), (0, k * s), (0, 0)))
             for k in range(ncopy)], axis=0)
        invcnt = jnp.asarray(_fold_inv_count(s))[None, :, :, None]
        tgrid = pltpu.PrefetchScalarGridSpec(
            num_scalar_prefetch=2,
            grid=(B,),
            in_specs=[
                pl.BlockSpec(shifts.shape,
                             lambda b, pos_r, idx_r: (0, 0, 0, 0)),
                pl.BlockSpec(invcnt.shape,
                             lambda b, pos_r, idx_r: (0, 0, 0, 0)),
            ],
            out_specs=pl.BlockSpec((1, 8 * s, 8 * s, C),
                                   lambda b, pos_r, idx_r: (b, 0, 0, 0)),
            scratch_shapes=[pltpu.VMEM((10 * s, 10 * s, C), F32)],
        )
        warped[s] = pl.pallas_call(
            functools.partial(_transfer_kernel, s=s),
            grid_spec=tgrid,
            out_shape=jax.ShapeDtypeStruct((B, 8 * s, 8 * s, C), F32),
        )(pos, idx_flat, shifts, invcnt)

    att8 = att[:, 0, :].reshape(B, 8, 8)
    outs = {}
    for s, C in ((1, C1), (2, C2), (4, C4)):
        att_up = jnp.repeat(jnp.repeat(att8, s, axis=1), s, axis=2)
        w = warped[s] * att_up[:, :, :, None]
        w = jnp.transpose(w, (0, 3, 1, 2))
        w = w.reshape(1, nb, nb, C, 8 * s, 8 * s)
        w = jnp.transpose(w, (0, 3, 1, 4, 2, 5)).reshape(
            1, C, Hr * s, Wr * s)
        outs[s] = w
    return (outs[4], outs[2], outs[1])
